# baseline (speedup 1.0000x reference)
import functools

import jax
import jax.numpy as jnp
from jax import lax
from jax.experimental import pallas as pl
from jax.experimental.pallas import tpu as pltpu
from jax.experimental.pallas import tpu_sc as plsc

N = 10000
E = 320000
DF = 128
DE = 16
H = 128
L = 3

NC = 2
NS = 16
NW = NC * NS
CHUNK = 128
CPW = 80
EP = NW * CHUNK * CPW
NCHUNK = EP // CHUNK
NP = 10112
ZR = NP // NS
OR_ = 624
TAIL = N - NS * OR_
GRP = 8
NGRP = CPW // GRP


def _sc_scatter_kernel():
    mesh = plsc.VectorSubcoreMesh(core_axis_name="c", subcore_axis_name="s")
    out_type = [jax.ShapeDtypeStruct((NC, N, H), jnp.float32)]
    scratch = [
        pltpu.VMEM_SHARED((NP, H), jnp.float32),
        pltpu.VMEM((GRP, CHUNK), jnp.int32),
        pltpu.VMEM((GRP, CHUNK), jnp.int32),
        pltpu.VMEM((CHUNK, H), jnp.float32),
        pltpu.VMEM((CHUNK, H), jnp.float32),
        pltpu.SemaphoreType.DMA,
        pltpu.SemaphoreType.DMA,
    ]

    def body(a_hbm, srcq, dstq, z128, out_s,
             s_acc, src_v, dst_v, rows_a, rows_b, sem_a, sem_b):
        c = lax.axis_index("c")
        s = lax.axis_index("s")
        wid = c * NS + s
        pltpu.sync_copy(z128, s_acc.at[pl.ds(s * ZR, ZR)])
        plsc.subcore_barrier()

        def group(g, _):
            base = wid * CPW + g * GRP
            pltpu.sync_copy(srcq.at[c, pl.ds(base, GRP)], src_v)
            pltpu.sync_copy(dstq.at[pl.ds(base, GRP)], dst_v)
            def fire(k, buf, sem):
                pltpu.async_copy(a_hbm.at[src_v.at[k, pl.ds(0, CHUNK // 2)]],
                                 buf.at[pl.ds(0, CHUNK // 2)], sem)
                pltpu.async_copy(a_hbm.at[src_v.at[k, pl.ds(CHUNK // 2, CHUNK // 2)]],
                                 buf.at[pl.ds(CHUNK // 2, CHUNK // 2)], sem)

            def drain(k, buf, sem):
                pltpu.make_async_copy(a_hbm.at[src_v.at[k, pl.ds(0, CHUNK // 2)]],
                                      buf.at[pl.ds(0, CHUNK // 2)], sem).wait()
                pltpu.make_async_copy(a_hbm.at[src_v.at[k, pl.ds(CHUNK // 2, CHUNK // 2)]],
                                      buf.at[pl.ds(CHUNK // 2, CHUNK // 2)], sem).wait()

            fire(0, rows_a, sem_a)
            for k in range(GRP):
                buf, sem = (rows_a, sem_a) if k % 2 == 0 else (rows_b, sem_b)
                nbuf, nsem = (rows_b, sem_b) if k % 2 == 0 else (rows_a, sem_a)
                drain(k, buf, sem)
                if k + 1 < GRP:
                    fire(k + 1, nbuf, nsem)
                pltpu.sync_copy(buf, s_acc.at[dst_v.at[k]], add=True)
            return 0

        lax.fori_loop(0, NGRP, group, 0)
        plsc.subcore_barrier()
        pltpu.sync_copy(s_acc.at[pl.ds(s * OR_, OR_)],
                        out_s.at[c, pl.ds(s * OR_, OR_)])

        @pl.when(s == NS - 1)
        def _():
            base = NS * OR_
            pltpu.sync_copy(s_acc.at[pl.ds(base, TAIL)],
                            out_s.at[c, pl.ds(base, TAIL)])

    return pl.kernel(body, out_type=out_type, mesh=mesh, scratch_types=scratch)


BR = 512
GRID = (N + BR - 1) // BR


BRE = 2048
EGRID = (E + BRE - 1) // BRE


def _tc_ewide_body(ea_ref, wedge_ref, o_ref):
    e16 = jnp.dot(ea_ref[...], wedge_ref[...], preferred_element_type=jnp.float32)
    ebf = e16.astype(jnp.bfloat16).astype(jnp.float32)
    o_ref[...] = jnp.concatenate(
        [ebf, jnp.ones((BRE, 1), jnp.float32),
         jnp.zeros((BRE, H - DE - 1), jnp.float32)], axis=1)


def _tc_ewide(edge_attr, wedge):
    return pl.pallas_call(
        _tc_ewide_body,
        grid=(EGRID,),
        in_specs=[
            pl.BlockSpec((BRE, DE), lambda i: (i, 0)),
            pl.BlockSpec((DE, DE), lambda i: (0, 0)),
        ],
        out_specs=pl.BlockSpec((BRE, H), lambda i: (i, 0)),
        out_shape=jax.ShapeDtypeStruct((E, H), jnp.float32),
    )(edge_attr, wedge)


def _tc_init_body(x_ref, wn_ref, wj_ref, h_ref, a_ref):
    h = jnp.dot(x_ref[...], wn_ref[...], preferred_element_type=jnp.float32)
    h_ref[...] = h
    a = jnp.dot(h, wj_ref[...], preferred_element_type=jnp.float32)
    a_ref[...] = jnp.broadcast_to(a[None], (NC, BR, H))


def _tc_init(x, w_node, wj0):
    return pl.pallas_call(
        _tc_init_body,
        grid=(GRID,),
        in_specs=[
            pl.BlockSpec((BR, DF), lambda i: (i, 0)),
            pl.BlockSpec((DF, H), lambda i: (0, 0)),
            pl.BlockSpec((H, H), lambda i: (0, 0)),
        ],
        out_specs=[
            pl.BlockSpec((BR, H), lambda i: (i, 0)),
            pl.BlockSpec((NC, BR, H), lambda i: (0, i, 0)),
        ],
        out_shape=[jax.ShapeDtypeStruct((N, H), jnp.float32),
                   jax.ShapeDtypeStruct((NC, N, H), jnp.float32)],
    )(x, w_node, wj0)


def _tc_update_body(compute_next, h_ref, sp_ref, ep_ref,
                    we_ref, wi_ref, bm_ref, wih_ref, whh_ref, bih_ref,
                    bhh_ref, wjn_ref, hn_ref, an_ref):
    h = h_ref[...]
    s = sp_ref[0] + sp_ref[1]
    e128 = ep_ref[0] + ep_ref[1]
    eagg = e128[:, :DE]
    deg = e128[:, DE:DE + 1]
    econ = jnp.dot(eagg, we_ref[...], preferred_element_type=jnp.float32,
                   precision=lax.Precision.HIGHEST)
    b = jnp.dot(h, wi_ref[...], preferred_element_type=jnp.float32)
    agg = s + econ + deg * (b + bm_ref[...])
    gi = jnp.dot(agg, wih_ref[...], preferred_element_type=jnp.float32) + bih_ref[...]
    gh = jnp.dot(h, whh_ref[...], preferred_element_type=jnp.float32) + bhh_ref[...]
    r = jax.nn.sigmoid(gi[:, :H] + gh[:, :H])
    z = jax.nn.sigmoid(gi[:, H:2 * H] + gh[:, H:2 * H])
    n = jnp.tanh(gi[:, 2 * H:] + r * gh[:, 2 * H:])
    hn = (1.0 - z) * n + z * h
    hn_ref[...] = hn
    if compute_next:
        an = jnp.dot(hn, wjn_ref[...], preferred_element_type=jnp.float32)
        an_ref[...] = jnp.broadcast_to(an[None], (NC, BR, H))


def _tc_update(compute_next, h, s_part, e_part, we, wi, bm,
               wih, whh, bih, bhh, wjn):
    body = functools.partial(_tc_update_body, compute_next)
    out_shape = [jax.ShapeDtypeStruct((N, H), jnp.float32),
                 jax.ShapeDtypeStruct((NC, N, H), jnp.float32)]
    return pl.pallas_call(
        body,
        grid=(GRID,),
        in_specs=[
            pl.BlockSpec((BR, H), lambda i: (i, 0)),
            pl.BlockSpec((NC, BR, H), lambda i: (0, i, 0)),
            pl.BlockSpec((NC, BR, H), lambda i: (0, i, 0)),
            pl.BlockSpec((DE, H), lambda i: (0, 0)),
            pl.BlockSpec((H, H), lambda i: (0, 0)),
            pl.BlockSpec((1, H), lambda i: (0, 0)),
            pl.BlockSpec((H, 3 * H), lambda i: (0, 0)),
            pl.BlockSpec((H, 3 * H), lambda i: (0, 0)),
            pl.BlockSpec((1, 3 * H), lambda i: (0, 0)),
            pl.BlockSpec((1, 3 * H), lambda i: (0, 0)),
            pl.BlockSpec((H, H), lambda i: (0, 0)),
        ],
        out_specs=[
            pl.BlockSpec((BR, H), lambda i: (i, 0)),
            pl.BlockSpec((NC, BR, H), lambda i: (0, i, 0)),
        ],
        out_shape=out_shape,
    )(h, s_part, e_part, we, wi, bm, wih, whh, bih, bhh, wjn)


def kernel(x, edge_index, edge_attr, W_node, W_edge, msg_W, msg_b,
           W_ih, W_hh, b_ih, b_hh):
    x = x.astype(jnp.float32)
    edge_attr = edge_attr.astype(jnp.float32)
    src = edge_index[0]
    dst = edge_index[1]

    pad = EP - E
    pad_ar = jnp.arange(pad, dtype=jnp.int32)
    src_pad = pad_ar % N
    dst_pad = N + pad_ar % (NP - N)
    src_q1 = jnp.concatenate([src, src_pad]).reshape(NCHUNK, CHUNK)
    src_q = jnp.stack([src_q1, src_q1 + N])
    dst_q = jnp.concatenate([dst, dst_pad]).reshape(NCHUNK, CHUNK)
    iota_q1 = jnp.minimum(jnp.arange(EP, dtype=jnp.int32), E - 1)
    iota_q1 = iota_q1.reshape(NCHUNK, CHUNK)
    iota_q = jnp.stack([iota_q1, iota_q1])
    z128 = jnp.zeros((ZR, H), jnp.float32)

    wj = [msg_W[l, :H] for l in range(L)]
    we = [msg_W[l, H:H + DE].astype(jnp.bfloat16).astype(jnp.float32)
          for l in range(L)]
    wi = [msg_W[l, H + DE:] for l in range(L)]
    bm = [msg_b[l].reshape(1, H) for l in range(L)]
    wih = [W_ih[l] for l in range(L)]
    whh = [W_hh[l] for l in range(L)]
    bih = [b_ih[l].reshape(1, 3 * H) for l in range(L)]
    bhh = [b_hh[l].reshape(1, 3 * H) for l in range(L)]

    sc_plain = _sc_scatter_kernel()

    h, a = _tc_init(x, W_node, wj[0])
    ea_wide = _tc_ewide(edge_attr, W_edge)

    (s_part,) = sc_plain(a.reshape(NC * N, H), src_q, dst_q, z128)
    ea_dep, _ = lax.optimization_barrier((ea_wide, s_part))
    (e_part,) = sc_plain(ea_dep, iota_q, dst_q, z128)

    for l in range(L):
        last = l == L - 1
        h, a = _tc_update(not last, h, s_part, e_part, we[l], wi[l],
                          bm[l], wih[l], whh[l], bih[l], bhh[l],
                          wj[min(l + 1, L - 1)])
        if not last:
            (s_part,) = sc_plain(a.reshape(NC * N, H), src_q, dst_q, z128)
    return h


# speedup vs baseline: 1.3036x; 1.3036x over previous
import functools

import jax
import jax.numpy as jnp
from jax import lax
from jax.experimental import pallas as pl
from jax.experimental.pallas import tpu as pltpu
from jax.experimental.pallas import tpu_sc as plsc

N = 10000
E = 320000
DF = 128
DE = 16
H = 128
L = 3

NC = 2
NS = 16
NW = NC * NS
CHUNK = 128
CPW = 80
EP = NW * CHUNK * CPW
NCHUNK = EP // CHUNK
NP = 10112
ZR = NP // NS
OR_ = 624
TAIL = N - NS * OR_
GRP = 8
NGRP = CPW // GRP


def _sc_scatter_kernel():
    mesh = plsc.VectorSubcoreMesh(core_axis_name="c", subcore_axis_name="s")
    out_type = [jax.ShapeDtypeStruct((NC, N, H), jnp.float32)]
    scratch = [
        pltpu.VMEM_SHARED((NP, H), jnp.float32),
        pltpu.VMEM((GRP, CHUNK), jnp.int32),
        pltpu.VMEM((GRP, CHUNK), jnp.int32),
        pltpu.VMEM((CHUNK, H), jnp.float32),
        pltpu.VMEM((CHUNK, H), jnp.float32),
        pltpu.SemaphoreType.DMA,
        pltpu.SemaphoreType.DMA,
    ]

    def body(a_hbm, srcq, dstq, z128, out_s,
             s_acc, src_v, dst_v, rows_a, rows_b, sem_a, sem_b):
        c = lax.axis_index("c")
        s = lax.axis_index("s")
        wid = c * NS + s
        pltpu.sync_copy(z128, s_acc.at[pl.ds(s * ZR, ZR)])
        plsc.subcore_barrier()

        def group(g, _):
            base = wid * CPW + g * GRP
            pltpu.sync_copy(srcq.at[c, pl.ds(base, GRP)], src_v)
            pltpu.sync_copy(dstq.at[pl.ds(base, GRP)], dst_v)
            def fire(k, buf, sem):
                pltpu.async_copy(a_hbm.at[src_v.at[k, pl.ds(0, CHUNK // 2)]],
                                 buf.at[pl.ds(0, CHUNK // 2)], sem)
                pltpu.async_copy(a_hbm.at[src_v.at[k, pl.ds(CHUNK // 2, CHUNK // 2)]],
                                 buf.at[pl.ds(CHUNK // 2, CHUNK // 2)], sem)

            def drain(k, buf, sem):
                pltpu.make_async_copy(a_hbm.at[src_v.at[k, pl.ds(0, CHUNK // 2)]],
                                      buf.at[pl.ds(0, CHUNK // 2)], sem).wait()
                pltpu.make_async_copy(a_hbm.at[src_v.at[k, pl.ds(CHUNK // 2, CHUNK // 2)]],
                                      buf.at[pl.ds(CHUNK // 2, CHUNK // 2)], sem).wait()

            fire(0, rows_a, sem_a)
            for k in range(GRP):
                buf, sem = (rows_a, sem_a) if k % 2 == 0 else (rows_b, sem_b)
                nbuf, nsem = (rows_b, sem_b) if k % 2 == 0 else (rows_a, sem_a)
                drain(k, buf, sem)
                if k + 1 < GRP:
                    fire(k + 1, nbuf, nsem)
                pltpu.sync_copy(buf, s_acc.at[dst_v.at[k]], add=True)
            return 0

        lax.fori_loop(0, NGRP, group, 0)
        plsc.subcore_barrier()
        pltpu.sync_copy(s_acc.at[pl.ds(s * OR_, OR_)],
                        out_s.at[c, pl.ds(s * OR_, OR_)])

        @pl.when(s == NS - 1)
        def _():
            base = NS * OR_
            pltpu.sync_copy(s_acc.at[pl.ds(base, TAIL)],
                            out_s.at[c, pl.ds(base, TAIL)])

    return pl.kernel(body, out_type=out_type, mesh=mesh, scratch_types=scratch)


BR = 512
GRID = (N + BR - 1) // BR


BRE = 2048
EGRID = EP // BRE


def _tc_ewide_body(ea_ref, wedge_ref, o_ref):
    e16 = jnp.dot(ea_ref[...], wedge_ref[...], preferred_element_type=jnp.float32)
    ebf = e16.astype(jnp.bfloat16).astype(jnp.float32)
    o_ref[...] = jnp.concatenate(
        [ebf, jnp.ones((BRE, 1), jnp.float32),
         jnp.zeros((BRE, H - DE - 1), jnp.float32)], axis=1)


def _tc_ewide(ea_q, wedge):
    return pl.pallas_call(
        _tc_ewide_body,
        grid=(EGRID,),
        in_specs=[
            pl.BlockSpec((BRE, DE), lambda i: (i, 0)),
            pl.BlockSpec((DE, DE), lambda i: (0, 0)),
        ],
        out_specs=pl.BlockSpec((BRE, H), lambda i: (i, 0)),
        out_shape=jax.ShapeDtypeStruct((EP, H), jnp.float32),
    )(ea_q, wedge)


def _tc_init_body(x_ref, wn_ref, wj_ref, h_ref, a_ref):
    h = jnp.dot(x_ref[...], wn_ref[...], preferred_element_type=jnp.float32)
    h_ref[...] = h
    a = jnp.dot(h, wj_ref[...], preferred_element_type=jnp.float32)
    a_ref[...] = jnp.broadcast_to(a[None], (NC, BR, H))


def _tc_init(x, w_node, wj0):
    return pl.pallas_call(
        _tc_init_body,
        grid=(GRID,),
        in_specs=[
            pl.BlockSpec((BR, DF), lambda i: (i, 0)),
            pl.BlockSpec((DF, H), lambda i: (0, 0)),
            pl.BlockSpec((H, H), lambda i: (0, 0)),
        ],
        out_specs=[
            pl.BlockSpec((BR, H), lambda i: (i, 0)),
            pl.BlockSpec((NC, BR, H), lambda i: (0, i, 0)),
        ],
        out_shape=[jax.ShapeDtypeStruct((N, H), jnp.float32),
                   jax.ShapeDtypeStruct((NC, N, H), jnp.float32)],
    )(x, w_node, wj0)


def _tc_update_body(compute_next, h_ref, sp_ref, ep_ref,
                    we_ref, wi_ref, bm_ref, wih_ref, whh_ref, bih_ref,
                    bhh_ref, wjn_ref, hn_ref, an_ref):
    h = h_ref[...]
    s = sp_ref[0] + sp_ref[1]
    e128 = ep_ref[0] + ep_ref[1]
    eagg = e128[:, :DE]
    deg = e128[:, DE:DE + 1]
    econ = jnp.dot(eagg, we_ref[...], preferred_element_type=jnp.float32,
                   precision=lax.Precision.HIGHEST)
    b = jnp.dot(h, wi_ref[...], preferred_element_type=jnp.float32)
    agg = s + econ + deg * (b + bm_ref[...])
    gi = jnp.dot(agg, wih_ref[...], preferred_element_type=jnp.float32) + bih_ref[...]
    gh = jnp.dot(h, whh_ref[...], preferred_element_type=jnp.float32) + bhh_ref[...]
    r = jax.nn.sigmoid(gi[:, :H] + gh[:, :H])
    z = jax.nn.sigmoid(gi[:, H:2 * H] + gh[:, H:2 * H])
    n = jnp.tanh(gi[:, 2 * H:] + r * gh[:, 2 * H:])
    hn = (1.0 - z) * n + z * h
    hn_ref[...] = hn
    if compute_next:
        an = jnp.dot(hn, wjn_ref[...], preferred_element_type=jnp.float32)
        an_ref[...] = jnp.broadcast_to(an[None], (NC, BR, H))


def _tc_update(compute_next, h, s_part, e_part, we, wi, bm,
               wih, whh, bih, bhh, wjn):
    body = functools.partial(_tc_update_body, compute_next)
    out_shape = [jax.ShapeDtypeStruct((N, H), jnp.float32),
                 jax.ShapeDtypeStruct((NC, N, H), jnp.float32)]
    return pl.pallas_call(
        body,
        grid=(GRID,),
        in_specs=[
            pl.BlockSpec((BR, H), lambda i: (i, 0)),
            pl.BlockSpec((NC, BR, H), lambda i: (0, i, 0)),
            pl.BlockSpec((NC, BR, H), lambda i: (0, i, 0)),
            pl.BlockSpec((DE, H), lambda i: (0, 0)),
            pl.BlockSpec((H, H), lambda i: (0, 0)),
            pl.BlockSpec((1, H), lambda i: (0, 0)),
            pl.BlockSpec((H, 3 * H), lambda i: (0, 0)),
            pl.BlockSpec((H, 3 * H), lambda i: (0, 0)),
            pl.BlockSpec((1, 3 * H), lambda i: (0, 0)),
            pl.BlockSpec((1, 3 * H), lambda i: (0, 0)),
            pl.BlockSpec((H, H), lambda i: (0, 0)),
        ],
        out_specs=[
            pl.BlockSpec((BR, H), lambda i: (i, 0)),
            pl.BlockSpec((NC, BR, H), lambda i: (0, i, 0)),
        ],
        out_shape=out_shape,
    )(h, s_part, e_part, we, wi, bm, wih, whh, bih, bhh, wjn)


def kernel(x, edge_index, edge_attr, W_node, W_edge, msg_W, msg_b,
           W_ih, W_hh, b_ih, b_hh):
    x = x.astype(jnp.float32)
    edge_attr = edge_attr.astype(jnp.float32)
    src = edge_index[0]
    dst = edge_index[1]

    pad = EP - E
    pad_ar = jnp.arange(pad, dtype=jnp.int32)
    src_pad = pad_ar % N
    dst_pad = N + pad_ar % (NP - N)
    src_q1 = jnp.concatenate([src, src_pad]).reshape(NCHUNK, CHUNK)
    src_q = jnp.stack([src_q1, src_q1 + N])
    dst_q = jnp.concatenate([dst, dst_pad]).reshape(NCHUNK, CHUNK)
    ea_q = jnp.concatenate([edge_attr, jnp.zeros((pad, DE), jnp.float32)])
    iota_q1 = jnp.arange(EP, dtype=jnp.int32).reshape(NCHUNK, CHUNK)
    iota_q = jnp.stack([iota_q1, iota_q1])
    z128 = jnp.zeros((ZR, H), jnp.float32)

    wj = [msg_W[l, :H] for l in range(L)]
    we = [msg_W[l, H:H + DE].astype(jnp.bfloat16).astype(jnp.float32)
          for l in range(L)]
    wi = [msg_W[l, H + DE:] for l in range(L)]
    bm = [msg_b[l].reshape(1, H) for l in range(L)]
    wih = [W_ih[l] for l in range(L)]
    whh = [W_hh[l] for l in range(L)]
    bih = [b_ih[l].reshape(1, 3 * H) for l in range(L)]
    bhh = [b_hh[l].reshape(1, 3 * H) for l in range(L)]

    sc_plain = _sc_scatter_kernel()

    h, a = _tc_init(x, W_node, wj[0])
    ea_wide = _tc_ewide(ea_q, W_edge)

    (s_part,) = sc_plain(a.reshape(NC * N, H), src_q, dst_q, z128)
    ea_dep, _ = lax.optimization_barrier((ea_wide, s_part))
    (e_part,) = sc_plain(ea_dep, iota_q, dst_q, z128)

    for l in range(L):
        last = l == L - 1
        h, a = _tc_update(not last, h, s_part, e_part, we[l], wi[l],
                          bm[l], wih[l], whh[l], bih[l], bhh[l],
                          wj[min(l + 1, L - 1)])
        if not last:
            (s_part,) = sc_plain(a.reshape(NC * N, H), src_q, dst_q, z128)
    return h
